# Initial kernel scaffold; baseline (speedup 1.0000x reference)
import jax
import jax.numpy as jnp
from jax.experimental import pallas as pl


def kernel(x, pos, edge_attr, diff_descriptors, params, edge_index, batch):
    raise NotImplementedError("write your pallas kernel here")


# speedup vs baseline: 1.7293x; 1.7293x over previous
import functools

import jax
import jax.numpy as jnp
from jax import lax
from jax.experimental import pallas as pl
from jax.experimental.pallas import tpu as pltpu
from jax.experimental.pallas import tpu_sc as plsc

_INTERP = False
_USE_SC = True

_NC, _NS, _L = 2, 16, 16
_NW = _NC * _NS


def _silu(x):
    return x * (1.0 / (1.0 + jnp.exp(-x)))


def _ceil_to(x, m):
    return (x + m - 1) // m * m


def _tc_linear(x, W, b=None, act=False, bm=512):
    M, K = x.shape
    Nn = W.shape[1]
    assert M % bm == 0, (M, bm)
    if b is None:
        b = jnp.zeros((1, Nn), jnp.float32)
    else:
        b = b.reshape(1, Nn)

    def body(x_ref, w_ref, b_ref, o_ref):
        y = jnp.dot(x_ref[...], w_ref[...], preferred_element_type=jnp.float32)
        y = y + b_ref[...]
        if act:
            y = _silu(y)
        o_ref[...] = y

    return pl.pallas_call(
        body,
        grid=(M // bm,),
        in_specs=[
            pl.BlockSpec((bm, K), lambda i: (i, 0)),
            pl.BlockSpec((K, Nn), lambda i: (0, 0)),
            pl.BlockSpec((1, Nn), lambda i: (0, 0)),
        ],
        out_specs=pl.BlockSpec((bm, Nn), lambda i: (i, 0)),
        out_shape=jax.ShapeDtypeStruct((M, Nn), jnp.float32),
        interpret=_INTERP,
    )(x, W, b)


def _tc_edge(g1, g2, eaC, ps, pd, w1d, b1, W2, b2, bm=512):
    M = g1.shape[0]
    H = g1.shape[1]

    def body(g1_ref, g2_ref, eaC_ref, ps_ref, pd_ref, w1d_ref, b1_ref,
             W2_ref, b2_ref, m_ref, unit_ref):
        rel = ps_ref[...] - pd_ref[...]
        s = jnp.sum(rel * rel, axis=1, keepdims=True)
        dist = jnp.sqrt(s)
        unit_ref[...] = rel / (dist + 1e-8)
        pre = (g1_ref[...] + g2_ref[...] + eaC_ref[...]
               + dist * w1d_ref[...] + b1_ref[...])
        u = _silu(pre)
        m = jnp.dot(u, W2_ref[...], preferred_element_type=jnp.float32)
        m_ref[...] = _silu(m + b2_ref[...])

    return pl.pallas_call(
        body,
        grid=(M // bm,),
        in_specs=[
            pl.BlockSpec((bm, H), lambda i: (i, 0)),
            pl.BlockSpec((bm, H), lambda i: (i, 0)),
            pl.BlockSpec((bm, H), lambda i: (i, 0)),
            pl.BlockSpec((bm, 16), lambda i: (i, 0)),
            pl.BlockSpec((bm, 16), lambda i: (i, 0)),
            pl.BlockSpec((1, H), lambda i: (0, 0)),
            pl.BlockSpec((1, H), lambda i: (0, 0)),
            pl.BlockSpec((H, H), lambda i: (0, 0)),
            pl.BlockSpec((1, H), lambda i: (0, 0)),
        ],
        out_specs=[
            pl.BlockSpec((bm, H), lambda i: (i, 0)),
            pl.BlockSpec((bm, 16), lambda i: (i, 0)),
        ],
        out_shape=[
            jax.ShapeDtypeStruct((M, H), jnp.float32),
            jax.ShapeDtypeStruct((M, 16), jnp.float32),
        ],
        interpret=_INTERP,
    )(g1, g2, eaC, ps, pd, w1d.reshape(1, H), b1.reshape(1, H), W2,
      b2.reshape(1, H))


def _tc_node(h, xn0, xn1, Wn1a, Wn1b, bn1, Wn2, bn2, C1, c1, C2_16, bm=1000):
    M, H = h.shape

    def body(h_ref, a_ref, b_ref, Wn1a_ref, Wn1b_ref, bn1_ref, Wn2_ref,
             bn2_ref, C1_ref, c1_ref, C2_ref, h2_ref, cw_ref):
        h_ = h_ref[...]
        xn = a_ref[...] + b_ref[...]
        u = _silu(jnp.dot(h_, Wn1a_ref[...], preferred_element_type=jnp.float32)
                  + jnp.dot(xn, Wn1b_ref[...], preferred_element_type=jnp.float32)
                  + bn1_ref[...])
        h2_ref[...] = h_ + jnp.dot(u, Wn2_ref[...],
                                   preferred_element_type=jnp.float32) + bn2_ref[...]
        t = _silu(jnp.dot(xn, C1_ref[...], preferred_element_type=jnp.float32)
                  + c1_ref[...])
        cw_ref[...] = jnp.dot(t, C2_ref[...], preferred_element_type=jnp.float32)

    full = lambda shape: pl.BlockSpec(shape, lambda i: (0, 0))
    row = lambda w: pl.BlockSpec((bm, w), lambda i: (i, 0))
    return pl.pallas_call(
        body,
        grid=(M // bm,),
        in_specs=[row(H), row(H), row(H), full((H, H)), full((H, H)),
                  full((1, H)), full((H, H)), full((1, H)), full((H, H)),
                  full((1, H)), full((H, 16))],
        out_specs=[row(H), row(16)],
        out_shape=[jax.ShapeDtypeStruct((M, H), jnp.float32),
                   jax.ShapeDtypeStruct((M, 16), jnp.float32)],
        interpret=_INTERP,
    )(h, xn0, xn1, Wn1a, Wn1b, bn1.reshape(1, -1), Wn2, bn2.reshape(1, -1),
      C1, c1.reshape(1, -1), C2_16)


def _tc_mul(a, b, bm=4096):
    M, W = a.shape
    bm = min(bm, M)

    def body(a_ref, b_ref, o_ref):
        o_ref[...] = a_ref[...] * b_ref[...]

    return pl.pallas_call(
        body,
        grid=(M // bm,),
        in_specs=[pl.BlockSpec((bm, W), lambda i: (i, 0))] * 2,
        out_specs=pl.BlockSpec((bm, W), lambda i: (i, 0)),
        out_shape=jax.ShapeDtypeStruct((M, W), jnp.float32),
        interpret=_INTERP,
    )(a, b)


def _tc_add3(a, b, c, bm=1000):
    M, W = a.shape

    def body(a_ref, b_ref, c_ref, o_ref):
        o_ref[...] = a_ref[...] + b_ref[...] + c_ref[...]

    return pl.pallas_call(
        body,
        grid=(M // bm,),
        in_specs=[pl.BlockSpec((bm, W), lambda i: (i, 0))] * 3,
        out_specs=pl.BlockSpec((bm, W), lambda i: (i, 0)),
        out_shape=jax.ShapeDtypeStruct((M, W), jnp.float32),
        interpret=_INTERP,
    )(a, b, c)


def _tc_head(spart, diffp, D1p, d1b, D2, d2b, P1a, P1b, p1b, P2, p2b, P3p, p3b):
    def body(s_ref, diff_ref, D1_ref, d1b_ref, D2_ref, d2b_ref, P1a_ref,
             P1b_ref, p1b_ref, P2_ref, p2b_ref, P3_ref, p3b_ref, o_ref):
        s = s_ref[0:16, :] + s_ref[16:32, :]
        gsum = s[0:8, 0:128]
        cnt = s[0:8, 128:129]
        g = gsum / jnp.maximum(cnt, 1.0)
        d = _silu(jnp.dot(diff_ref[...], D1_ref[...],
                          preferred_element_type=jnp.float32) + d1b_ref[...])
        d = jnp.dot(d, D2_ref[...], preferred_element_type=jnp.float32) + d2b_ref[...]
        p = _silu(jnp.dot(g, P1a_ref[...], preferred_element_type=jnp.float32)
                  + jnp.dot(d, P1b_ref[...], preferred_element_type=jnp.float32)
                  + p1b_ref[...])
        q = _silu(jnp.dot(p, P2_ref[...], preferred_element_type=jnp.float32)
                  + p2b_ref[...])
        o_ref[...] = jnp.dot(q, P3_ref[...],
                             preferred_element_type=jnp.float32) + p3b_ref[...]

    return pl.pallas_call(
        body,
        out_shape=jax.ShapeDtypeStruct((8, 16), jnp.float32),
        interpret=_INTERP,
    )(spart, diffp, D1p, d1b.reshape(1, -1), D2, d2b.reshape(1, -1),
      P1a, P1b, p1b.reshape(1, -1), P2, p2b.reshape(1, -1), P3p,
      p3b.reshape(1, -1))


@functools.cache
def _sc_gather_fn(T, D, E_pad):
    epw = E_pad // _NW
    assert epw % 1024 == 0
    outer = epw // 1024
    mesh = plsc.VectorSubcoreMesh(core_axis_name="c", subcore_axis_name="s")

    @functools.partial(
        pl.kernel,
        out_type=jax.ShapeDtypeStruct((E_pad, D), jnp.float32),
        mesh=mesh,
        compiler_params=pltpu.CompilerParams(use_tc_tiling_on_sc=False),
        scratch_types=[
            pltpu.VMEM((8, 128), jnp.int32),
            pltpu.VMEM((512, D), jnp.float32),
            pltpu.SemaphoreType.DMA,
        ],
    )
    def k(table_hbm, idx_hbm, out_hbm, idx_v, rows_v, sem):
        wid = lax.axis_index("c") * _NS + lax.axis_index("s")
        base = wid * epw

        def step(i, carry):
            off = pl.multiple_of(base + i * 1024, 1024)
            pltpu.sync_copy(idx_hbm.at[pl.ds(pl.multiple_of(off // 128, 8), 8)], idx_v)
            for half in range(2):
                cps = []
                for j in range(4):
                    cps.append(pltpu.async_copy(
                        table_hbm.at[idx_v.at[half * 4 + j]],
                        rows_v.at[pl.ds(j * 128, 128)], sem))
                for cp in cps:
                    cp.wait()
                pltpu.sync_copy(rows_v, out_hbm.at[pl.ds(off + half * 512, 512)])
            return carry

        lax.fori_loop(0, outer, step, 0)

    return k


def _sc_gather(table, idx2):
    T, D = table.shape
    E_pad = idx2.shape[0] * 128
    if not _USE_SC:
        return table[idx2.reshape(-1)]
    return _sc_gather_fn(T, D, E_pad)(table, idx2)


@functools.cache
def _sc_scatter_fn(Tp, D, E_pad):
    epw = E_pad // _NW
    assert epw % 1024 == 0
    outer = epw // 1024
    rows_pt = Tp // _NS
    assert Tp % 128 == 0
    mesh = plsc.VectorSubcoreMesh(core_axis_name="c", subcore_axis_name="s")

    @functools.partial(
        pl.kernel,
        out_type=jax.ShapeDtypeStruct((_NC, Tp, D), jnp.float32),
        mesh=mesh,
        compiler_params=pltpu.CompilerParams(use_tc_tiling_on_sc=False),
        scratch_types=[
            pltpu.VMEM((8, 128), jnp.int32),
            pltpu.VMEM((128, D), jnp.float32),
            pltpu.VMEM_SHARED((Tp, D), jnp.float32),
        ],
    )
    def k(vals_hbm, idx_hbm, zeros_hbm, out_hbm, idx_v, vals_v, acc_sh):
        cid = lax.axis_index("c")
        sid = lax.axis_index("s")
        wid = cid * _NS + sid
        r0 = pl.multiple_of(sid * rows_pt, 8)
        pltpu.sync_copy(zeros_hbm.at[pl.ds(r0, rows_pt)],
                        acc_sh.at[pl.ds(r0, rows_pt)])
        plsc.subcore_barrier()
        base = wid * epw

        def step(i, carry):
            off = pl.multiple_of(base + i * 1024, 1024)
            pltpu.sync_copy(idx_hbm.at[pl.ds(pl.multiple_of(off // 128, 8), 8)], idx_v)
            for j in range(8):
                pltpu.sync_copy(vals_hbm.at[pl.ds(off + j * 128, 128)],
                                vals_v)
                pltpu.sync_copy(vals_v, acc_sh.at[idx_v.at[j]], add=True)
            return carry

        lax.fori_loop(0, outer, step, 0)
        plsc.subcore_barrier()
        pltpu.sync_copy(acc_sh.at[pl.ds(r0, rows_pt)],
                        out_hbm.at[cid, pl.ds(r0, rows_pt)])

    return k


def _sc_scatter(vals, idx2, Tp):
    E_pad, D = vals.shape
    if not _USE_SC:
        flat = jnp.zeros((Tp, D), jnp.float32).at[idx2.reshape(-1)].add(vals)
        return jnp.stack([flat, jnp.zeros_like(flat)])
    zeros = jnp.zeros((Tp, D), jnp.float32)
    return _sc_scatter_fn(Tp, D, E_pad)(vals, idx2, zeros)


def kernel(x, pos, edge_attr, diff_descriptors, params, edge_index, batch):
    N, NODE_F = x.shape
    E = edge_index.shape[1]
    H = params["node_emb"]["W"].shape[1]
    B = diff_descriptors.shape[0]

    E_pad = _ceil_to(E, _NW * 1024)
    N_pad = _ceil_to(N, _NW * 1024)
    Tp = _ceil_to(N + 1, 128)
    DUMP = Tp - 1

    src = edge_index[0]
    dst = edge_index[1]
    src_g = jnp.pad(src, (0, E_pad - E)).reshape(-1, 128)
    dst_g = jnp.pad(dst, (0, E_pad - E)).reshape(-1, 128)
    src_s = jnp.pad(src, (0, E_pad - E), constant_values=DUMP).reshape(-1, 128)
    dst_s = jnp.pad(dst, (0, E_pad - E), constant_values=DUMP).reshape(-1, 128)
    batch_s = jnp.pad(batch, (0, N_pad - N), constant_values=B).reshape(-1, 128)

    Np = _ceil_to(N, 1000)
    xp = jnp.pad(x, ((0, Np - N), (0, 0)))
    h = _tc_linear(xp, params["node_emb"]["W"], params["node_emb"]["b"],
                   bm=1000)[:N]
    eap = jnp.pad(edge_attr, ((0, E_pad - E), (0, 0)))
    ea = _tc_linear(eap, params["edge_emb"]["W"], params["edge_emb"]["b"])

    pos16 = jnp.pad(pos, ((0, 0), (0, 16 - pos.shape[1])))

    hpad = lambda a: jnp.pad(a, ((0, Np - N), (0, 0)))

    for lp in params["layers"]:
        W1 = lp["msg1"]["W"]
        b1 = lp["msg1"]["b"]
        hA = _tc_linear(hpad(h), W1[H:2 * H], bm=1000)[:N]
        hB = _tc_linear(hpad(h), W1[0:H], bm=1000)[:N]
        eaC = _tc_linear(ea, W1[2 * H:3 * H])
        w1d = W1[3 * H]

        g1 = _sc_gather(hB, dst_g)
        g2 = _sc_gather(hA, src_g)
        ps = _sc_gather(pos16, src_g)
        pd = _sc_gather(pos16, dst_g)

        m, unit = _tc_edge(g1, g2, eaC, ps, pd, w1d, b1,
                           lp["msg2"]["W"], lp["msg2"]["b"])

        xnp = _sc_scatter(m, dst_s, Tp)
        xn0 = xnp[0, :N]
        xn1 = xnp[1, :N]

        Wn1 = lp["node1"]["W"]
        C2_16 = jnp.tile(lp["coord2"]["W"], (1, 16))
        h2, cw16 = _tc_node(hpad(h), hpad(xn0), hpad(xn1),
                            Wn1[0:H], Wn1[H:2 * H], lp["node1"]["b"],
                            lp["node2"]["W"], lp["node2"]["b"],
                            lp["coord1"]["W"], lp["coord1"]["b"], C2_16)
        h = h2[:N]
        cw16 = cw16[:N]

        cws = _sc_gather(cw16, src_g)
        delta = _tc_mul(unit, cws)
        dpart = _sc_scatter(delta, src_s, Tp)
        pos16 = _tc_add3(hpad(pos16), hpad(dpart[0, :N]),
                         hpad(dpart[1, :N]))[:N]

    hcat = jnp.concatenate([h, jnp.ones((N, 16), jnp.float32)], axis=1)
    hcat = jnp.pad(hcat, ((0, N_pad - N), (0, 0)))
    spart = _sc_scatter(hcat, batch_s, 128)
    spart = jnp.concatenate([spart[0, :16], spart[1, :16]], axis=0)

    diffp = jnp.pad(diff_descriptors, ((0, 0), (0, 16 - 11)))
    D1p = jnp.pad(params["diff1"]["W"], ((0, 16 - 11), (0, 0)))
    P1 = params["pred1"]["W"]
    P3p = jnp.pad(params["pred3"]["W"], ((0, 0), (0, 16 - 3)))
    p3b = jnp.pad(params["pred3"]["b"], (0, 16 - 3))
    out = _tc_head(spart, diffp, D1p, params["diff1"]["b"],
                   params["diff2"]["W"], params["diff2"]["b"],
                   P1[0:H], P1[H:], params["pred1"]["b"],
                   params["pred2"]["W"], params["pred2"]["b"], P3p, p3b)
    return out[:B, :3]


# speedup vs baseline: 2.0954x; 1.2117x over previous
import functools

import jax
import jax.numpy as jnp
from jax import lax
from jax.experimental import pallas as pl
from jax.experimental.pallas import tpu as pltpu
from jax.experimental.pallas import tpu_sc as plsc

_INTERP = False
_USE_SC = True
_USE_G4 = True
_USE_CD = True

_NC, _NS, _L = 2, 16, 16
_NW = _NC * _NS


def _silu(x):
    return x * jax.nn.sigmoid(x)


def _dot(a, b):
    return jnp.dot(a, b, preferred_element_type=jnp.float32)


def _ceil_to(x, m):
    return (x + m - 1) // m * m


def _tc_linear(x, W, b=None, act=False, bm=512):
    M, K = x.shape
    Nn = W.shape[1]
    assert M % bm == 0, (M, bm)
    if b is None:
        b = jnp.zeros((1, Nn), jnp.float32)
    else:
        b = b.reshape(1, Nn)

    def body(x_ref, w_ref, b_ref, o_ref):
        y = _dot(x_ref[...], w_ref[...])
        y = y + b_ref[...]
        if act:
            y = _silu(y)
        o_ref[...] = y

    return pl.pallas_call(
        body,
        grid=(M // bm,),
        in_specs=[
            pl.BlockSpec((bm, K), lambda i: (i, 0)),
            pl.BlockSpec((K, Nn), lambda i: (0, 0)),
            pl.BlockSpec((1, Nn), lambda i: (0, 0)),
        ],
        out_specs=pl.BlockSpec((bm, Nn), lambda i: (i, 0)),
        out_shape=jax.ShapeDtypeStruct((M, Nn), jnp.float32),
        interpret=_INTERP,
    )(x, W, b)


def _tc_edge(hd, hs, ea, rel_in, W1, b1, W2, b2, bm=512):
    M, H = hd.shape

    def body(hd_ref, hs_ref, ea_ref, rel_ref, W1_ref, b1_ref,
             W2_ref, b2_ref, m_ref, unit_ref):
        rel = rel_ref[...]
        s = jnp.sum(rel * rel, axis=1, keepdims=True)
        dist = jnp.sqrt(s)
        unit_ref[...] = rel / (dist + 1e-8)
        cat = jnp.concatenate(
            [hd_ref[...], hs_ref[...], ea_ref[...], dist], axis=1)
        u = _silu(_dot(cat, W1_ref[...]) + b1_ref[...])
        m = _dot(u, W2_ref[...])
        m_ref[...] = _silu(m + b2_ref[...])

    return pl.pallas_call(
        body,
        grid=(M // bm,),
        in_specs=[
            pl.BlockSpec((bm, H), lambda i: (i, 0)),
            pl.BlockSpec((bm, H), lambda i: (i, 0)),
            pl.BlockSpec((bm, H), lambda i: (i, 0)),
            pl.BlockSpec((bm, 16), lambda i: (i, 0)),
            pl.BlockSpec((3 * H + 1, H), lambda i: (0, 0)),
            pl.BlockSpec((1, H), lambda i: (0, 0)),
            pl.BlockSpec((H, H), lambda i: (0, 0)),
            pl.BlockSpec((1, H), lambda i: (0, 0)),
        ],
        out_specs=[
            pl.BlockSpec((bm, H), lambda i: (i, 0)),
            pl.BlockSpec((bm, 16), lambda i: (i, 0)),
        ],
        out_shape=[
            jax.ShapeDtypeStruct((M, H), jnp.float32),
            jax.ShapeDtypeStruct((M, 16), jnp.float32),
        ],
        interpret=_INTERP,
    )(hd, hs, ea, rel_in, W1, b1.reshape(1, H), W2, b2.reshape(1, H))


def _tc_node(h, xn0, xn1, Wn1, bn1, Wn2, bn2, C1, c1, C2_16, bm=1000):
    M, H = h.shape

    def body(h_ref, a_ref, b_ref, Wn1_ref, bn1_ref, Wn2_ref,
             bn2_ref, C1_ref, c1_ref, C2_ref, h2_ref, cw_ref):
        h_ = h_ref[...]
        xn = a_ref[...] + b_ref[...]
        u = _silu(_dot(jnp.concatenate([h_, xn], axis=1), Wn1_ref[...])
                  + bn1_ref[...])
        h2_ref[...] = h_ + _dot(u, Wn2_ref[...]) + bn2_ref[...]
        t = _silu(_dot(xn, C1_ref[...])
                  + c1_ref[...])
        cw_ref[...] = _dot(t, C2_ref[...])

    full = lambda shape: pl.BlockSpec(shape, lambda i: (0, 0))
    row = lambda w: pl.BlockSpec((bm, w), lambda i: (i, 0))
    return pl.pallas_call(
        body,
        grid=(M // bm,),
        in_specs=[row(H), row(H), row(H), full((2 * H, H)),
                  full((1, H)), full((H, H)), full((1, H)), full((H, H)),
                  full((1, H)), full((H, 16))],
        out_specs=[row(H), row(16)],
        out_shape=[jax.ShapeDtypeStruct((M, H), jnp.float32),
                   jax.ShapeDtypeStruct((M, 16), jnp.float32)],
        interpret=_INTERP,
    )(h, xn0, xn1, Wn1, bn1.reshape(1, -1), Wn2, bn2.reshape(1, -1),
      C1, c1.reshape(1, -1), C2_16)


def _tc_mul(a, b, bm=4096):
    M, W = a.shape
    bm = min(bm, M)

    def body(a_ref, b_ref, o_ref):
        o_ref[...] = a_ref[...] * b_ref[...]

    return pl.pallas_call(
        body,
        grid=(M // bm,),
        in_specs=[pl.BlockSpec((bm, W), lambda i: (i, 0))] * 2,
        out_specs=pl.BlockSpec((bm, W), lambda i: (i, 0)),
        out_shape=jax.ShapeDtypeStruct((M, W), jnp.float32),
        interpret=_INTERP,
    )(a, b)


def _tc_add3(a, b, c, bm=1000):
    M, W = a.shape

    def body(a_ref, b_ref, c_ref, o_ref):
        o_ref[...] = a_ref[...] + b_ref[...] + c_ref[...]

    return pl.pallas_call(
        body,
        grid=(M // bm,),
        in_specs=[pl.BlockSpec((bm, W), lambda i: (i, 0))] * 3,
        out_specs=pl.BlockSpec((bm, W), lambda i: (i, 0)),
        out_shape=jax.ShapeDtypeStruct((M, W), jnp.float32),
        interpret=_INTERP,
    )(a, b, c)


def _tc_head(spart, diffp, D1p, d1b, D2, d2b, P1, p1b, P2, p2b, P3p, p3b):
    def body(s_ref, diff_ref, D1_ref, d1b_ref, D2_ref, d2b_ref, P1_ref,
             p1b_ref, P2_ref, p2b_ref, P3_ref, p3b_ref, o_ref):
        s = s_ref[0:16, :] + s_ref[16:32, :]
        gsum = s[0:8, 0:128]
        cnt = s[0:8, 128:129]
        g = gsum / jnp.maximum(cnt, 1.0)
        d = _silu(_dot(diff_ref[...], D1_ref[...]) + d1b_ref[...])
        d = _dot(d, D2_ref[...]) + d2b_ref[...]
        p = _silu(_dot(jnp.concatenate([g, d], axis=1), P1_ref[...])
                  + p1b_ref[...])
        q = _silu(_dot(p, P2_ref[...])
                  + p2b_ref[...])
        o_ref[...] = _dot(q, P3_ref[...]) + p3b_ref[...]

    return pl.pallas_call(
        body,
        out_shape=jax.ShapeDtypeStruct((8, 16), jnp.float32),
        interpret=_INTERP,
    )(spart, diffp, D1p, d1b.reshape(1, -1), D2, d2b.reshape(1, -1),
      P1, p1b.reshape(1, -1), P2, p2b.reshape(1, -1), P3p,
      p3b.reshape(1, -1))


@functools.cache
def _sc_gather_fn(T, D, E_pad):
    epw = E_pad // _NW
    assert epw % 1024 == 0
    outer = epw // 1024
    mesh = plsc.VectorSubcoreMesh(core_axis_name="c", subcore_axis_name="s")

    @functools.partial(
        pl.kernel,
        out_type=jax.ShapeDtypeStruct((E_pad, D), jnp.float32),
        mesh=mesh,
        compiler_params=pltpu.CompilerParams(use_tc_tiling_on_sc=False),
        scratch_types=[
            pltpu.VMEM((8, 128), jnp.int32),
            pltpu.VMEM((512, D), jnp.float32),
            pltpu.SemaphoreType.DMA,
        ],
    )
    def k(table_hbm, idx_hbm, out_hbm, idx_v, rows_v, sem):
        wid = lax.axis_index("c") * _NS + lax.axis_index("s")
        base = wid * epw

        def step(i, carry):
            off = pl.multiple_of(base + i * 1024, 1024)
            pltpu.sync_copy(idx_hbm.at[pl.ds(pl.multiple_of(off // 128, 8), 8)], idx_v)
            for half in range(2):
                cps = []
                for j in range(4):
                    cps.append(pltpu.async_copy(
                        table_hbm.at[idx_v.at[half * 4 + j]],
                        rows_v.at[pl.ds(j * 128, 128)], sem))
                for cp in cps:
                    cp.wait()
                pltpu.sync_copy(rows_v, out_hbm.at[pl.ds(off + half * 512, 512)])
            return carry

        lax.fori_loop(0, outer, step, 0)

    return k


def _sc_gather(table, idx2):
    T, D = table.shape
    E_pad = idx2.shape[0] * 128
    if not _USE_SC:
        return table[idx2.reshape(-1)]
    return _sc_gather_fn(T, D, E_pad)(table, idx2)


@functools.cache
def _sc_scatter_fn(Tp, D, E_pad):
    epw = E_pad // _NW
    assert epw % 1024 == 0
    outer = epw // 1024
    rows_pt = Tp // _NS
    assert Tp % 128 == 0
    mesh = plsc.VectorSubcoreMesh(core_axis_name="c", subcore_axis_name="s")

    @functools.partial(
        pl.kernel,
        out_type=jax.ShapeDtypeStruct((_NC, Tp, D), jnp.float32),
        mesh=mesh,
        compiler_params=pltpu.CompilerParams(use_tc_tiling_on_sc=False),
        scratch_types=[
            pltpu.VMEM((8, 128), jnp.int32),
            pltpu.VMEM((128, D), jnp.float32),
            pltpu.VMEM_SHARED((Tp, D), jnp.float32),
        ],
    )
    def k(vals_hbm, idx_hbm, zeros_hbm, out_hbm, idx_v, vals_v, acc_sh):
        cid = lax.axis_index("c")
        sid = lax.axis_index("s")
        wid = cid * _NS + sid
        r0 = pl.multiple_of(sid * rows_pt, 8)
        pltpu.sync_copy(zeros_hbm.at[pl.ds(r0, rows_pt)],
                        acc_sh.at[pl.ds(r0, rows_pt)])
        plsc.subcore_barrier()
        base = wid * epw

        def step(i, carry):
            off = pl.multiple_of(base + i * 1024, 1024)
            pltpu.sync_copy(idx_hbm.at[pl.ds(pl.multiple_of(off // 128, 8), 8)], idx_v)
            for j in range(8):
                pltpu.sync_copy(vals_hbm.at[pl.ds(off + j * 128, 128)],
                                vals_v)
                pltpu.sync_copy(vals_v, acc_sh.at[idx_v.at[j]], add=True)
            return carry

        lax.fori_loop(0, outer, step, 0)
        plsc.subcore_barrier()
        pltpu.sync_copy(acc_sh.at[pl.ds(r0, rows_pt)],
                        out_hbm.at[cid, pl.ds(r0, rows_pt)])

    return k


def _sc_scatter(vals, idx2, Tp):
    E_pad, D = vals.shape
    if not _USE_SC:
        flat = jnp.zeros((Tp, D), jnp.float32).at[idx2.reshape(-1)].add(vals)
        return jnp.stack([flat, jnp.zeros_like(flat)])
    zeros = jnp.zeros((Tp, D), jnp.float32)
    return _sc_scatter_fn(Tp, D, E_pad)(vals, idx2, zeros)


@functools.cache
def _sc_gather4_fn(T, H, E_pad):
    epw = E_pad // _NW
    assert epw % 1024 == 0
    outer = epw // 1024
    mesh = plsc.VectorSubcoreMesh(core_axis_name="c", subcore_axis_name="s")

    @functools.partial(
        pl.kernel,
        out_type=[jax.ShapeDtypeStruct((E_pad, H), jnp.float32),
                  jax.ShapeDtypeStruct((E_pad, H), jnp.float32),
                  jax.ShapeDtypeStruct((E_pad, 16), jnp.float32)],
        mesh=mesh,
        compiler_params=pltpu.CompilerParams(use_tc_tiling_on_sc=False),
        scratch_types=[
            pltpu.VMEM((8, 128), jnp.int32),
            pltpu.VMEM((8, 128), jnp.int32),
            pltpu.VMEM((256, H), jnp.float32),
            pltpu.VMEM((256, H), jnp.float32),
            pltpu.VMEM((256, 16), jnp.float32),
            pltpu.VMEM((256, 16), jnp.float32),
            pltpu.SemaphoreType.DMA,
        ],
    )
    def k(hB_hbm, hA_hbm, pos_hbm, dst_hbm, src_hbm, g1_hbm, g2_hbm, rel_hbm,
          idxd, idxs, r1, r2, p1, p2, sem):
        wid = lax.axis_index("c") * _NS + lax.axis_index("s")
        base = wid * epw

        def step(i, carry):
            off = pl.multiple_of(base + i * 1024, 1024)
            row0 = pl.multiple_of(off // 128, 8)
            pltpu.sync_copy(dst_hbm.at[pl.ds(row0, 8)], idxd)
            pltpu.sync_copy(src_hbm.at[pl.ds(row0, 8)], idxs)
            for q in range(4):
                cps = []
                for j in range(2):
                    r = q * 2 + j
                    dsj = pl.ds(j * 128, 128)
                    cps.append(pltpu.async_copy(hB_hbm.at[idxd.at[r]],
                                                r1.at[dsj], sem))
                    cps.append(pltpu.async_copy(hA_hbm.at[idxs.at[r]],
                                                r2.at[dsj], sem))
                    cps.append(pltpu.async_copy(pos_hbm.at[idxs.at[r]],
                                                p1.at[dsj], sem))
                    cps.append(pltpu.async_copy(pos_hbm.at[idxd.at[r]],
                                                p2.at[dsj], sem))
                for cp in cps:
                    cp.wait()

                def sub8(rr, c):
                    for u in range(8):
                        row = rr * 8 + u
                        p1[row, :] = p1[row, :] - p2[row, :]
                    return c

                lax.fori_loop(0, 32, sub8, 0)
                oq = pl.multiple_of(off + q * 256, 256)
                pltpu.sync_copy(r1, g1_hbm.at[pl.ds(oq, 256)])
                pltpu.sync_copy(r2, g2_hbm.at[pl.ds(oq, 256)])
                pltpu.sync_copy(p1, rel_hbm.at[pl.ds(oq, 256)])
            return carry

        lax.fori_loop(0, outer, step, 0)

    return k


def _sc_gather4(hB, hA, pos16, dst2, src2):
    T, H = hB.shape
    E_pad = dst2.shape[0] * 128
    if not (_USE_SC and _USE_G4):
        dst = dst2.reshape(-1)
        src = src2.reshape(-1)
        return hB[dst], hA[src], pos16[src] - pos16[dst]
    return _sc_gather4_fn(T, H, E_pad)(hB, hA, pos16, dst2, src2)


@functools.cache
def _sc_coord_fn(Tp, E_pad):
    epw = E_pad // _NW
    assert epw % 1024 == 0
    outer = epw // 1024
    rows_pt = Tp // _NS
    mesh = plsc.VectorSubcoreMesh(core_axis_name="c", subcore_axis_name="s")

    @functools.partial(
        pl.kernel,
        out_type=jax.ShapeDtypeStruct((_NC, Tp, 16), jnp.float32),
        mesh=mesh,
        compiler_params=pltpu.CompilerParams(use_tc_tiling_on_sc=False),
        scratch_types=[
            pltpu.VMEM((8, 128), jnp.int32),
            pltpu.VMEM((256, 16), jnp.float32),
            pltpu.VMEM((256, 16), jnp.float32),
            pltpu.VMEM_SHARED((Tp, 16), jnp.float32),
            pltpu.SemaphoreType.DMA,
        ],
    )
    def k(cw_hbm, unit_hbm, src_hbm, zeros_hbm, out_hbm, idxs, un, cw,
          acc_sh, sem):
        cid = lax.axis_index("c")
        sid = lax.axis_index("s")
        wid = cid * _NS + sid
        r0 = pl.multiple_of(sid * rows_pt, 8)
        pltpu.sync_copy(zeros_hbm.at[pl.ds(r0, rows_pt)],
                        acc_sh.at[pl.ds(r0, rows_pt)])
        plsc.subcore_barrier()
        base = wid * epw

        def step(i, carry):
            off = pl.multiple_of(base + i * 1024, 1024)
            row0 = pl.multiple_of(off // 128, 8)
            pltpu.sync_copy(src_hbm.at[pl.ds(row0, 8)], idxs)
            for q in range(4):
                oq = pl.multiple_of(off + q * 256, 256)
                pltpu.sync_copy(unit_hbm.at[pl.ds(oq, 256)], un)
                cps = []
                for j in range(2):
                    cps.append(pltpu.async_copy(
                        cw_hbm.at[idxs.at[q * 2 + j]],
                        cw.at[pl.ds(j * 128, 128)], sem))
                for cp in cps:
                    cp.wait()

                def mul8(rr, c):
                    for u in range(8):
                        row = rr * 8 + u
                        un[row, :] = un[row, :] * cw[row, :]
                    return c

                lax.fori_loop(0, 32, mul8, 0)
                for j in range(2):
                    pltpu.sync_copy(un.at[pl.ds(j * 128, 128)],
                                    acc_sh.at[idxs.at[q * 2 + j]], add=True)
            return carry

        lax.fori_loop(0, outer, step, 0)
        plsc.subcore_barrier()
        pltpu.sync_copy(acc_sh.at[pl.ds(r0, rows_pt)],
                        out_hbm.at[cid, pl.ds(r0, rows_pt)])

    return k


def _sc_coord(cwp, unit, src2, Tp):
    E_pad = unit.shape[0]
    if not (_USE_SC and _USE_CD):
        src = src2.reshape(-1)
        delta = unit * cwp[src]
        flat = jnp.zeros((Tp, 16), jnp.float32).at[src].add(delta)
        return jnp.stack([flat, jnp.zeros_like(flat)])
    zeros = jnp.zeros((Tp, 16), jnp.float32)
    return _sc_coord_fn(Tp, E_pad)(cwp, unit, src2, zeros)


def kernel(x, pos, edge_attr, diff_descriptors, params, edge_index, batch):
    N, NODE_F = x.shape
    E = edge_index.shape[1]
    H = params["node_emb"]["W"].shape[1]
    B = diff_descriptors.shape[0]

    E_pad = _ceil_to(E, _NW * 1024)
    N_pad = _ceil_to(N, _NW * 1024)
    Tp = _ceil_to(N + 1, 128)
    DUMP = Tp - 1

    src = edge_index[0]
    dst = edge_index[1]
    src_g = jnp.pad(src, (0, E_pad - E)).reshape(-1, 128)
    dst_g = jnp.pad(dst, (0, E_pad - E)).reshape(-1, 128)
    src_s = jnp.pad(src, (0, E_pad - E), constant_values=DUMP).reshape(-1, 128)
    dst_s = jnp.pad(dst, (0, E_pad - E), constant_values=DUMP).reshape(-1, 128)
    batch_s = jnp.pad(batch, (0, N_pad - N), constant_values=B).reshape(-1, 128)

    Np = _ceil_to(N, 1000)
    xp = jnp.pad(x, ((0, Np - N), (0, 0)))
    h = _tc_linear(xp, params["node_emb"]["W"], params["node_emb"]["b"],
                   bm=1000)[:N]
    eap = jnp.pad(edge_attr, ((0, E_pad - E), (0, 0)))
    ea = _tc_linear(eap, params["edge_emb"]["W"], params["edge_emb"]["b"])

    pos16 = jnp.pad(pos, ((0, 0), (0, 16 - pos.shape[1])))

    hpad = lambda a: jnp.pad(a, ((0, Np - N), (0, 0)))

    for lp in params["layers"]:
        hd, hs, rel = _sc_gather4(h, h, pos16, dst_g, src_g)

        m, unit = _tc_edge(hd, hs, ea, rel, lp["msg1"]["W"], lp["msg1"]["b"],
                           lp["msg2"]["W"], lp["msg2"]["b"])

        xnp = _sc_scatter(m, dst_s, Tp)
        xn0 = xnp[0, :N]
        xn1 = xnp[1, :N]

        C2_16 = jnp.tile(lp["coord2"]["W"], (1, 16))
        h2, cw16 = _tc_node(hpad(h), hpad(xn0), hpad(xn1),
                            lp["node1"]["W"], lp["node1"]["b"],
                            lp["node2"]["W"], lp["node2"]["b"],
                            lp["coord1"]["W"], lp["coord1"]["b"], C2_16)
        h = h2[:N]
        cw16 = cw16[:N]

        cwp = jnp.pad(cw16, ((0, Tp - N), (0, 0)))
        dpart = _sc_coord(cwp, unit, src_s, Tp)
        pos16 = _tc_add3(hpad(pos16), hpad(dpart[0, :N]),
                         hpad(dpart[1, :N]))[:N]

    hcat = jnp.concatenate([h, jnp.ones((N, 16), jnp.float32)], axis=1)
    hcat = jnp.pad(hcat, ((0, N_pad - N), (0, 0)))
    spart = _sc_scatter(hcat, batch_s, 128)
    spart = jnp.concatenate([spart[0, :16], spart[1, :16]], axis=0)

    diffp = jnp.pad(diff_descriptors, ((0, 0), (0, 16 - 11)))
    D1p = jnp.pad(params["diff1"]["W"], ((0, 16 - 11), (0, 0)))
    P1 = params["pred1"]["W"]
    P3p = jnp.pad(params["pred3"]["W"], ((0, 0), (0, 16 - 3)))
    p3b = jnp.pad(params["pred3"]["b"], (0, 16 - 3))
    out = _tc_head(spart, diffp, D1p, params["diff1"]["b"],
                   params["diff2"]["W"], params["diff2"]["b"],
                   P1, params["pred1"]["b"],
                   params["pred2"]["W"], params["pred2"]["b"], P3p, p3b)
    return out[:B, :3]
